# baseline (speedup 1.0000x reference)
import jax
import jax.numpy as jnp
from jax.experimental import pallas as pl
from jax.experimental.pallas import tpu as pltpu

L, B, D, H = 256, 64, 512, 512
H3 = 3 * H
C = 8
NB = L // C


def _gru(gx, gh, bias, hp):
    r = jax.nn.sigmoid(gx[:, :H] + bias[:, :H] + gh[:, :H])
    z = jax.nn.sigmoid(gx[:, H:2 * H] + bias[:, H:2 * H] + gh[:, H:2 * H])
    n = jnp.tanh(gx[:, 2 * H:] + bias[:, 2 * H:] + r * gh[:, 2 * H:])
    return (1.0 - z) * n + z * hp


def _both_kernel(heads_sref, embr_ref, embf_ref, wdt_ref, udt_ref, bdt_ref,
                 wtd_ref, utd_ref, btd_ref, outdt_ref, outtd_ref,
                 cs_ref, hid_ref, gxdt_ref, gxtd_ref, hpn_ref):
    i = pl.program_id(0)

    @pl.when(i == 0)
    def _():
        cs_ref[...] = jnp.zeros_like(cs_ref)
        hid_ref[L] = jnp.zeros((B, H), jnp.bfloat16)

    gxdt_ref[...] = jnp.dot(embr_ref[...].reshape(C * B, D), wdt_ref[...],
                            preferred_element_type=jnp.float32)
    gxtd_ref[...] = jnp.dot(embf_ref[...].reshape(C * B, D), wtd_ref[...],
                            preferred_element_type=jnp.float32)

    base_dt = (NB - 1 - i) * C
    base_td = i * C
    bias_dt = bdt_ref[...]
    bias_td = btd_ref[...]
    u_dt = udt_ref[...]
    u_td = utd_ref[...]

    def step(j, carry):
        l_dt = C - 1 - j
        t_dt = base_dt + l_dt
        t_td = base_td + j
        for b in range(B):
            p = heads_sref[b, t_td]
            hpn_ref[b, :] = hid_ref[p, b, :]
        hp_dt = cs_ref[t_dt].astype(jnp.float32)
        gh_dt = jnp.dot(hp_dt, u_dt, preferred_element_type=jnp.float32)
        hp_td = hpn_ref[...].astype(jnp.float32)
        gh_td = jnp.dot(hp_td, u_td, preferred_element_type=jnp.float32)
        gx_dt = gxdt_ref[pl.ds(l_dt * B, B), :]
        gx_td = gxtd_ref[pl.ds(j * B, B), :]
        h_dt = _gru(gx_dt, gh_dt, bias_dt, hp_dt)
        h_td = _gru(gx_td, gh_td, bias_td, hp_td)
        h_td_bf = h_td.astype(jnp.bfloat16)
        h_dt_bf = h_dt.astype(jnp.bfloat16)
        outdt_ref[l_dt] = h_dt_bf
        outtd_ref[j] = h_td_bf
        hid_ref[t_td] = h_td_bf
        for b in range(B):
            p = heads_sref[b, t_dt]
            cs_ref[p, b, :] = cs_ref[p, b, :] + h_dt_bf[b, :]
        return carry

    jax.lax.fori_loop(0, C, step, 0)


def kernel(input, heads, W_dt, U_dt, b_dt, W_td, U_td, b_td):
    heads_i = heads.astype(jnp.int32)
    rev = lambda i, hr: (NB - 1 - i, 0, 0)
    fwd = lambda i, hr: (i, 0, 0)
    full2 = lambda i, hr: (0, 0)
    spec = pltpu.PrefetchScalarGridSpec(
        num_scalar_prefetch=1,
        grid=(NB,),
        in_specs=[
            pl.BlockSpec((C, B, D), rev),
            pl.BlockSpec((C, B, D), fwd),
            pl.BlockSpec((D, H3), full2),
            pl.BlockSpec((H, H3), full2),
            pl.BlockSpec((1, H3), full2),
            pl.BlockSpec((D, H3), full2),
            pl.BlockSpec((H, H3), full2),
            pl.BlockSpec((1, H3), full2),
        ],
        out_specs=[
            pl.BlockSpec((C, B, H), rev),
            pl.BlockSpec((C, B, H), fwd),
        ],
        scratch_shapes=[
            pltpu.VMEM((L + 1, B, H), jnp.bfloat16),
            pltpu.VMEM((L + 1, B, H), jnp.bfloat16),
            pltpu.VMEM((C * B, H3), jnp.float32),
            pltpu.VMEM((C * B, H3), jnp.float32),
            pltpu.VMEM((B, H), jnp.bfloat16),
        ],
    )
    dt_hid, td_hid = pl.pallas_call(
        _both_kernel,
        grid_spec=spec,
        out_shape=[jax.ShapeDtypeStruct((L, B, H), jnp.bfloat16),
                   jax.ShapeDtypeStruct((L, B, H), jnp.bfloat16)],
        compiler_params=pltpu.CompilerParams(
            dimension_semantics=("arbitrary",),
            vmem_limit_bytes=63 * 1024 * 1024),
    )(heads_i, input, input, W_dt, U_dt, b_dt.reshape(1, H3),
      W_td, U_td, b_td.reshape(1, H3))
    outputs = jnp.concatenate([dt_hid, td_hid], axis=2).transpose(1, 0, 2)
    outputs = outputs.astype(jnp.float32)
    output_t = jnp.concatenate([dt_hid[0], td_hid[L - 1]],
                               axis=1)[None].astype(jnp.float32)
    return outputs, output_t


# speedup vs baseline: 1.0615x; 1.0615x over previous
import jax
import jax.numpy as jnp
from jax.experimental import pallas as pl
from jax.experimental.pallas import tpu as pltpu

L, B, D, H = 256, 64, 512, 512
H3 = 3 * H
C = 8
NB = L // C


def _gru(gx, gh, bias, hp):
    r = jax.nn.sigmoid(gx[:, :H] + bias[:, :H] + gh[:, :H])
    z = jax.nn.sigmoid(gx[:, H:2 * H] + bias[:, H:2 * H] + gh[:, H:2 * H])
    n = jnp.tanh(gx[:, 2 * H:] + bias[:, 2 * H:] + r * gh[:, 2 * H:])
    return (1.0 - z) * n + z * hp


def _both_kernel(heads_sref, embr_ref, embf_ref, wdt_ref, udt_ref, bdt_ref,
                 wtd_ref, utd_ref, btd_ref, outdt_ref, outtd_ref,
                 cs_ref, hid_ref, gxdt_ref, gxtd_ref, hpn_ref):
    i = pl.program_id(0)

    @pl.when(i == 0)
    def _():
        cs_ref[...] = jnp.zeros_like(cs_ref)
        hid_ref[L] = jnp.zeros((B, H), jnp.bfloat16)

    gxdt_ref[...] = jnp.dot(embr_ref[...].reshape(C * B, D), wdt_ref[...],
                            preferred_element_type=jnp.float32)
    gxtd_ref[...] = jnp.dot(embf_ref[...].reshape(C * B, D), wtd_ref[...],
                            preferred_element_type=jnp.float32)

    base_dt = (NB - 1 - i) * C
    base_td = i * C
    bias_dt = bdt_ref[...]
    bias_td = btd_ref[...]
    u_dt = udt_ref[...]
    u_td = utd_ref[...]

    for j in range(C):
        l_dt = C - 1 - j
        t_dt = base_dt + l_dt
        t_td = base_td + j
        for b in range(B):
            p = heads_sref[b, t_td]
            hpn_ref[b, :] = hid_ref[p, b, :]
        hp_dt = cs_ref[t_dt].astype(jnp.float32)
        gh_dt = jnp.dot(hp_dt, u_dt, preferred_element_type=jnp.float32)
        hp_td = hpn_ref[...].astype(jnp.float32)
        gh_td = jnp.dot(hp_td, u_td, preferred_element_type=jnp.float32)
        gx_dt = gxdt_ref[pl.ds(l_dt * B, B), :]
        gx_td = gxtd_ref[pl.ds(j * B, B), :]
        h_dt = _gru(gx_dt, gh_dt, bias_dt, hp_dt)
        h_td = _gru(gx_td, gh_td, bias_td, hp_td)
        h_td_bf = h_td.astype(jnp.bfloat16)
        h_dt_bf = h_dt.astype(jnp.bfloat16)
        outdt_ref[l_dt] = h_dt_bf
        outtd_ref[j] = h_td_bf
        hid_ref[t_td] = h_td_bf
        for b in range(B):
            p = heads_sref[b, t_dt]
            cs_ref[p, b, :] = cs_ref[p, b, :] + h_dt_bf[b, :]


def kernel(input, heads, W_dt, U_dt, b_dt, W_td, U_td, b_td):
    heads_i = heads.astype(jnp.int32)
    rev = lambda i, hr: (NB - 1 - i, 0, 0)
    fwd = lambda i, hr: (i, 0, 0)
    full2 = lambda i, hr: (0, 0)
    spec = pltpu.PrefetchScalarGridSpec(
        num_scalar_prefetch=1,
        grid=(NB,),
        in_specs=[
            pl.BlockSpec((C, B, D), rev),
            pl.BlockSpec((C, B, D), fwd),
            pl.BlockSpec((D, H3), full2),
            pl.BlockSpec((H, H3), full2),
            pl.BlockSpec((1, H3), full2),
            pl.BlockSpec((D, H3), full2),
            pl.BlockSpec((H, H3), full2),
            pl.BlockSpec((1, H3), full2),
        ],
        out_specs=[
            pl.BlockSpec((C, B, H), rev),
            pl.BlockSpec((C, B, H), fwd),
        ],
        scratch_shapes=[
            pltpu.VMEM((L + 1, B, H), jnp.bfloat16),
            pltpu.VMEM((L + 1, B, H), jnp.bfloat16),
            pltpu.VMEM((C * B, H3), jnp.float32),
            pltpu.VMEM((C * B, H3), jnp.float32),
            pltpu.VMEM((B, H), jnp.bfloat16),
        ],
    )
    dt_hid, td_hid = pl.pallas_call(
        _both_kernel,
        grid_spec=spec,
        out_shape=[jax.ShapeDtypeStruct((L, B, H), jnp.bfloat16),
                   jax.ShapeDtypeStruct((L, B, H), jnp.bfloat16)],
        compiler_params=pltpu.CompilerParams(
            dimension_semantics=("arbitrary",),
            vmem_limit_bytes=63 * 1024 * 1024),
    )(heads_i, input, input, W_dt, U_dt, b_dt.reshape(1, H3),
      W_td, U_td, b_td.reshape(1, H3))
    outputs = jnp.concatenate([dt_hid, td_hid], axis=2).transpose(1, 0, 2)
    outputs = outputs.astype(jnp.float32)
    output_t = jnp.concatenate([dt_hid[0], td_hid[L - 1]],
                               axis=1)[None].astype(jnp.float32)
    return outputs, output_t


# speedup vs baseline: 1.0847x; 1.0219x over previous
import jax
import jax.numpy as jnp
from jax.experimental import pallas as pl
from jax.experimental.pallas import tpu as pltpu

L, B, D, H = 256, 64, 512, 512
H3 = 3 * H
C = 8
NB = L // C


def _gru(gx, gh, bias, hp):
    r = jax.nn.sigmoid(gx[:, :H] + bias[:, :H] + gh[:, :H])
    z = jax.nn.sigmoid(gx[:, H:2 * H] + bias[:, H:2 * H] + gh[:, H:2 * H])
    n = jnp.tanh(gx[:, 2 * H:] + bias[:, 2 * H:] + r * gh[:, 2 * H:])
    return (1.0 - z) * n + z * hp


def _both_kernel(heads_sref, embr_ref, embf_ref, wdt_ref, udt_ref, bdt_ref,
                 wtd_ref, utd_ref, btd_ref, outdt_ref, outtd_ref,
                 cs_ref, hid_ref, gxdt_ref, gxtd_ref, hpn_ref):
    i = pl.program_id(0)

    @pl.when(i == 0)
    def _():
        cs_ref[...] = jnp.zeros_like(cs_ref)
        hid_ref[pl.ds(L * 16, 16)] = jnp.zeros((16, 16, 128), jnp.bfloat16)

    gxdt_ref[...] = jnp.dot(embr_ref[...].reshape(C * B, D), wdt_ref[...],
                            preferred_element_type=jnp.float32)
    gxtd_ref[...] = jnp.dot(embf_ref[...].reshape(C * B, D), wtd_ref[...],
                            preferred_element_type=jnp.float32)

    base_dt = (NB - 1 - i) * C
    base_td = i * C
    bias_dt = bdt_ref[...]
    bias_td = btd_ref[...]
    u_dt = udt_ref[...]
    u_td = utd_ref[...]

    for j in range(C):
        l_dt = C - 1 - j
        t_dt = base_dt + l_dt
        t_td = base_td + j
        for b in range(B):
            p = heads_sref[b, t_td]
            hpn_ref[b] = hid_ref[p * 16 + b // 4, pl.ds(4 * (b % 4), 4), :]
        hp_dt = cs_ref[pl.ds(t_dt * 16, 16)].reshape(B, H).astype(jnp.float32)
        gh_dt = jnp.dot(hp_dt, u_dt, preferred_element_type=jnp.float32)
        hp_td = hpn_ref[...].reshape(B, H).astype(jnp.float32)
        gh_td = jnp.dot(hp_td, u_td, preferred_element_type=jnp.float32)
        gx_dt = gxdt_ref[pl.ds(l_dt * B, B), :]
        gx_td = gxtd_ref[pl.ds(j * B, B), :]
        h_dt = _gru(gx_dt, gh_dt, bias_dt, hp_dt)
        h_td = _gru(gx_td, gh_td, bias_td, hp_td)
        h_td_bf = h_td.astype(jnp.bfloat16)
        h_dt_bf = h_dt.astype(jnp.bfloat16)
        outdt_ref[l_dt] = h_dt_bf
        outtd_ref[j] = h_td_bf
        hid_ref[pl.ds(t_td * 16, 16)] = h_td_bf.reshape(16, 16, 128)
        hpk = h_dt_bf.reshape(B, 4, 128)
        for b in range(B):
            p = heads_sref[b, t_dt]
            r = p * 16 + b // 4
            sl = pl.ds(4 * (b % 4), 4)
            cs_ref[r, sl, :] = cs_ref[r, sl, :] + hpk[b]


def kernel(input, heads, W_dt, U_dt, b_dt, W_td, U_td, b_td):
    heads_i = heads.astype(jnp.int32)
    rev = lambda i, hr: (NB - 1 - i, 0, 0)
    fwd = lambda i, hr: (i, 0, 0)
    full2 = lambda i, hr: (0, 0)
    spec = pltpu.PrefetchScalarGridSpec(
        num_scalar_prefetch=1,
        grid=(NB,),
        in_specs=[
            pl.BlockSpec((C, B, D), rev),
            pl.BlockSpec((C, B, D), fwd),
            pl.BlockSpec((D, H3), full2),
            pl.BlockSpec((H, H3), full2),
            pl.BlockSpec((1, H3), full2),
            pl.BlockSpec((D, H3), full2),
            pl.BlockSpec((H, H3), full2),
            pl.BlockSpec((1, H3), full2),
        ],
        out_specs=[
            pl.BlockSpec((C, B, H), rev),
            pl.BlockSpec((C, B, H), fwd),
        ],
        scratch_shapes=[
            pltpu.VMEM(((L + 1) * 16, 16, 128), jnp.bfloat16),
            pltpu.VMEM(((L + 1) * 16, 16, 128), jnp.bfloat16),
            pltpu.VMEM((C * B, H3), jnp.float32),
            pltpu.VMEM((C * B, H3), jnp.float32),
            pltpu.VMEM((B, 4, 128), jnp.bfloat16),
        ],
    )
    dt_hid, td_hid = pl.pallas_call(
        _both_kernel,
        grid_spec=spec,
        out_shape=[jax.ShapeDtypeStruct((L, B, H), jnp.bfloat16),
                   jax.ShapeDtypeStruct((L, B, H), jnp.bfloat16)],
        compiler_params=pltpu.CompilerParams(
            dimension_semantics=("arbitrary",),
            vmem_limit_bytes=63 * 1024 * 1024),
    )(heads_i, input, input, W_dt, U_dt, b_dt.reshape(1, H3),
      W_td, U_td, b_td.reshape(1, H3))
    outputs = jnp.concatenate([dt_hid, td_hid], axis=2).transpose(1, 0, 2)
    outputs = outputs.astype(jnp.float32)
    output_t = jnp.concatenate([dt_hid[0], td_hid[L - 1]],
                               axis=1)[None].astype(jnp.float32)
    return outputs, output_t
